# baseline (speedup 1.0000x reference)
import functools

import jax
import jax.numpy as jnp
from jax import lax
from jax.experimental import pallas as pl
from jax.experimental.pallas import tpu as pltpu
from jax.experimental.pallas import tpu_sc as plsc

N = 50000
C = 128
E = 600000
CH = 32
NCK = C // CH
R = 51200
EPAD = 614400
EROWS = EPAD // 128
SROWS = EROWS // 16
HM = 384
NOP = SROWS * 128 // HM
MACB = 6
NMAC = SROWS // MACB
TPR = R // 16
FB = 640
NFB = TPR // FB
ZB = 64
NZB = TPR // ZB
BN = 1000

_mesh = plsc.VectorSubcoreMesh(
    core_axis_name="c", subcore_axis_name="s", num_cores=2, num_subcores=16)


@functools.partial(
    pl.kernel,
    out_type=jax.ShapeDtypeStruct((2, NCK, R, CH), jnp.float32),
    mesh=_mesh,
    scratch_types=[
        pltpu.VMEM((2 * HM,), jnp.int32),
        pltpu.VMEM((2 * HM,), jnp.int32),
        pltpu.VMEM((2 * HM, CH), jnp.float32),
        pltpu.VMEM_SHARED((R, CH), jnp.float32),
        pltpu.SemaphoreType.DMA,
        pltpu.SemaphoreType.DMA,
        pltpu.SemaphoreType.DMA,
        pltpu.SemaphoreType.DMA,
    ],
    compiler_params=pltpu.CompilerParams(use_tc_tiling_on_sc=False),
)
def _spmm(x_flat, src4_all, dst_all, zeros_hbm, out,
          sidx, didx, rows, accum, g0, g1, s0, s1):
    ci = lax.axis_index("c")
    si = lax.axis_index("s")
    base_edge = si * SROWS * 128
    arow0 = si * TPR

    def half(ref, p):
        return ref.at[pl.ds(p * HM, HM)]

    def drain(sem, p):
        pltpu.make_async_copy(x_flat.at[pl.ds(0, HM)], half(rows, p),
                              sem).wait()

    for et in range(2):
        for k in range(2):
            ck = ci * 2 + k

            def load_idx(m, p):
                e0 = base_edge + m * HM
                pltpu.sync_copy(src4_all.at[et, ck, pl.ds(e0, HM)],
                                half(sidx, p))
                pltpu.sync_copy(dst_all.at[et, pl.ds(e0, HM)], half(didx, p))

            def fire_gather(p, sem):
                pltpu.async_copy(x_flat.at[half(sidx, p)], half(rows, p), sem)

            def fire_scatter(p, sem):
                pltpu.async_copy(half(rows, p), accum.at[half(didx, p)], sem,
                                 add=True)

            pltpu.sync_copy(zeros_hbm, rows)
            for z in range(4):
                pltpu.sync_copy(rows, accum.at[pl.ds(arow0 + z * 2 * HM,
                                                     2 * HM)])
            pltpu.sync_copy(rows.at[pl.ds(0, TPR - 8 * HM)],
                            accum.at[pl.ds(arow0 + 8 * HM, TPR - 8 * HM)])
            plsc.subcore_barrier()

            load_idx(0, 0)
            fire_gather(0, g0)
            pltpu.async_copy(x_flat.at[pl.ds(0, HM)], half(rows, 1), s1)

            def blk(i, carry):
                m = 2 * i
                drain(s1, 1)
                load_idx(m + 1, 1)
                fire_gather(1, g1)
                drain(g0, 0)
                fire_scatter(0, s0)
                drain(g1, 1)
                fire_scatter(1, s1)
                drain(s0, 0)
                load_idx(m + 2, 0)
                fire_gather(0, g0)
                return carry

            lax.fori_loop(0, NOP // 2 - 1, blk, 0)
            drain(s1, 1)
            load_idx(NOP - 1, 1)
            fire_gather(1, g1)
            drain(g0, 0)
            fire_scatter(0, s0)
            drain(g1, 1)
            fire_scatter(1, s1)
            drain(s0, 0)
            drain(s1, 1)
            plsc.subcore_barrier()
            for z in range(NFB):
                r0 = arow0 + z * FB
                pltpu.sync_copy(accum.at[pl.ds(r0, FB)], rows.at[pl.ds(0, FB)])
                pltpu.sync_copy(rows.at[pl.ds(0, FB)],
                                out.at[et, ck, pl.ds(r0, FB)])


@functools.partial(
    pl.kernel,
    out_type=jax.ShapeDtypeStruct((2, R, 16), jnp.float32),
    mesh=_mesh,
    scratch_types=[
        pltpu.VMEM((MACB * 128,), jnp.int32),
        pltpu.VMEM((MACB * 128, 16), jnp.float32),
        pltpu.VMEM((FB, 16), jnp.float32),
        pltpu.VMEM_SHARED((R, 16), jnp.float32),
    ],
    compiler_params=pltpu.CompilerParams(use_tc_tiling_on_sc=False),
)
def _count(dst_all, ones_hbm, zeros_hbm, out, didx, ones_v, zbuf, accum):
    ci = lax.axis_index("c")
    si = lax.axis_index("s")
    base_edge = si * SROWS * 128
    arow0 = si * TPR
    pltpu.sync_copy(ones_hbm, ones_v)
    pltpu.sync_copy(zeros_hbm, zbuf)
    for z in range(NFB):
        pltpu.sync_copy(zbuf.at[pl.ds(0, FB)],
                        accum.at[pl.ds(arow0 + z * FB, FB)])
    plsc.subcore_barrier()

    def blk(b, carry):
        e0 = base_edge + b * (MACB * 128)
        pltpu.sync_copy(dst_all.at[ci, pl.ds(e0, MACB * 128)], didx)
        pltpu.sync_copy(ones_v, accum.at[didx], add=True)
        return carry

    lax.fori_loop(0, NMAC, blk, 0)
    plsc.subcore_barrier()
    for z in range(NFB):
        r0 = arow0 + z * FB
        pltpu.sync_copy(accum.at[pl.ds(r0, FB)], zbuf)
        pltpu.sync_copy(zbuf, out.at[ci, pl.ds(r0, FB)])


def _xr_body(x_ref, wr_ref, o_ref):
    o_ref[0] = lax.dot_general(x_ref[0], wr_ref[0],
                               dimension_numbers=(((1,), (1,)), ((), ())),
                               preferred_element_type=jnp.float32)


def _xr(x, wr):
    return pl.pallas_call(
        _xr_body,
        grid=(2, N // BN),
        in_specs=[
            pl.BlockSpec((1, BN, C), lambda t, i: (t, i, 0)),
            pl.BlockSpec((1, C, C), lambda t, i: (t, 0, 0)),
        ],
        out_specs=pl.BlockSpec((1, BN, C), lambda t, i: (t, i, 0)),
        out_shape=jax.ShapeDtypeStruct((2, N, C), jnp.float32),
    )(x, wr)


def _tc_body(agg_ref, cnt_ref, xr_ref, wl_ref, b_ref, g_ref, be_ref,
             o_ref):
    cnt = jnp.maximum(cnt_ref[0, :, 0:1], 1.0)
    acc = jnp.zeros((BN, C), jnp.float32)
    for ck in range(NCK):
        acc = acc + lax.dot_general(
            agg_ref[0, ck], wl_ref[0, :, ck * CH:(ck + 1) * CH],
            dimension_numbers=(((1,), (1,)), ((), ())),
            preferred_element_type=jnp.float32)
    y = acc / cnt + b_ref[0] + xr_ref[0]
    mu = jnp.mean(y, axis=-1, keepdims=True)
    yc = y - mu
    var = jnp.mean(yc * yc, axis=-1, keepdims=True)
    z = yc * lax.rsqrt(var + 1e-5) * g_ref[0] + be_ref[0]
    o_ref[0] = jnp.maximum(z, 0.0)


def _tc_layer(agg, cnt, xr, wl, bb, g, be):
    return pl.pallas_call(
        _tc_body,
        grid=(2, N // BN),
        in_specs=[
            pl.BlockSpec((1, NCK, BN, CH), lambda t, i: (t, 0, i, 0)),
            pl.BlockSpec((1, BN, 16), lambda t, i: (t, i, 0)),
            pl.BlockSpec((1, BN, C), lambda t, i: (t, i, 0)),
            pl.BlockSpec((1, C, C), lambda t, i: (t, 0, 0)),
            pl.BlockSpec((1, 1, C), lambda t, i: (t, 0, 0)),
            pl.BlockSpec((1, 1, C), lambda t, i: (t, 0, 0)),
            pl.BlockSpec((1, 1, C), lambda t, i: (t, 0, 0)),
        ],
        out_specs=pl.BlockSpec((1, BN, C), lambda t, i: (t, i, 0)),
        out_shape=jax.ShapeDtypeStruct((2, N, C), jnp.float32),
    )(agg, cnt, xr, wl, bb, g, be)


def _prep_edges(ei_u2i, ei_i2u):
    def one(ei, stack_off):
        srcp = jnp.concatenate(
            [ei[0], jnp.zeros((EPAD - E,), jnp.int32)])
        dstp = jnp.concatenate(
            [ei[1], jnp.full((EPAD - E,), N, jnp.int32)])
        s4 = (srcp + stack_off) * NCK
        s4 = s4[None, :] + jnp.arange(NCK, dtype=jnp.int32)[:, None]
        return s4, dstp

    s0, d0 = one(ei_u2i, N)
    s1, d1 = one(ei_i2u, 0)
    return jnp.stack([s0, s1]), jnp.stack([d0, d1])


def kernel(x_user, x_item, edge_index_u2i, edge_index_i2u, params):
    src4_all, dst_all = _prep_edges(edge_index_u2i, edge_index_i2u)
    ones16 = jnp.ones((MACB * 128, 16), jnp.float32)
    zeros16 = jnp.zeros((FB, 16), jnp.float32)
    zeros32 = jnp.zeros((2 * HM, CH), jnp.float32)
    cnt = _count(dst_all, ones16, zeros16)
    x = jnp.stack([x_item, x_user])
    for lp in params['layers']:
        wl = jnp.stack([lp['u2i']['W_l'], lp['i2u']['W_l']])
        bb = jnp.stack([lp['u2i']['b'], lp['i2u']['b']])[:, None, :]
        wr = jnp.stack([lp['u2i']['W_r'], lp['i2u']['W_r']])
        g = jnp.stack([lp['ln_item']['g'], lp['ln_user']['g']])[:, None, :]
        be = jnp.stack([lp['ln_item']['b'], lp['ln_user']['b']])[:, None, :]
        xr = _xr(x, wr)
        agg = _spmm(x.reshape(2 * N * NCK, CH), src4_all, dst_all, zeros32)
        x = _tc_layer(agg, cnt, xr, wl, bb, g, be)
    return x[1], x[0]


# speedup vs baseline: 1.0367x; 1.0367x over previous
import functools

import jax
import jax.numpy as jnp
from jax import lax
from jax.experimental import pallas as pl
from jax.experimental.pallas import tpu as pltpu
from jax.experimental.pallas import tpu_sc as plsc

N = 50000
C = 128
E = 600000
CH = 32
NCK = C // CH
R = 51200
EPAD = 614400
EROWS = EPAD // 128
SROWS = EROWS // 16
HM = 384
NOP = SROWS * 128 // HM
MACB = 6
NMAC = SROWS // MACB
TPR = R // 16
FB = 640
NFB = TPR // FB
ZB = 64
NZB = TPR // ZB
BN = 2000

_mesh = plsc.VectorSubcoreMesh(
    core_axis_name="c", subcore_axis_name="s", num_cores=2, num_subcores=16)


@functools.partial(
    pl.kernel,
    out_type=jax.ShapeDtypeStruct((2, NCK, R, CH), jnp.float32),
    mesh=_mesh,
    scratch_types=[
        pltpu.VMEM((2 * HM,), jnp.int32),
        pltpu.VMEM((2 * HM,), jnp.int32),
        pltpu.VMEM((2 * HM, CH), jnp.float32),
        pltpu.VMEM_SHARED((R, CH), jnp.float32),
        pltpu.SemaphoreType.DMA,
        pltpu.SemaphoreType.DMA,
        pltpu.SemaphoreType.DMA,
        pltpu.SemaphoreType.DMA,
    ],
    compiler_params=pltpu.CompilerParams(use_tc_tiling_on_sc=False),
)
def _spmm(x_flat, src4_all, dst_all, zeros_hbm, out,
          sidx, didx, rows, accum, g0, g1, s0, s1):
    ci = lax.axis_index("c")
    si = lax.axis_index("s")
    base_edge = si * SROWS * 128
    arow0 = si * TPR

    def half(ref, p):
        return ref.at[pl.ds(p * HM, HM)]

    def drain(sem, p):
        pltpu.make_async_copy(x_flat.at[pl.ds(0, HM)], half(rows, p),
                              sem).wait()

    for et in range(2):
        for k in range(2):
            ck = ci * 2 + k

            def load_idx(m, p):
                e0 = base_edge + m * HM
                pltpu.sync_copy(src4_all.at[et, ck, pl.ds(e0, HM)],
                                half(sidx, p))
                pltpu.sync_copy(dst_all.at[et, pl.ds(e0, HM)], half(didx, p))

            def fire_gather(p, sem):
                pltpu.async_copy(x_flat.at[half(sidx, p)], half(rows, p), sem)

            def fire_scatter(p, sem):
                pltpu.async_copy(half(rows, p), accum.at[half(didx, p)], sem,
                                 add=True)

            pltpu.sync_copy(zeros_hbm, rows)
            for z in range(4):
                pltpu.sync_copy(rows, accum.at[pl.ds(arow0 + z * 2 * HM,
                                                     2 * HM)])
            pltpu.sync_copy(rows.at[pl.ds(0, TPR - 8 * HM)],
                            accum.at[pl.ds(arow0 + 8 * HM, TPR - 8 * HM)])
            plsc.subcore_barrier()

            load_idx(0, 0)
            fire_gather(0, g0)
            pltpu.async_copy(x_flat.at[pl.ds(0, HM)], half(rows, 1), s1)

            def blk(i, carry):
                m = 2 * i
                drain(s1, 1)
                load_idx(m + 1, 1)
                fire_gather(1, g1)
                drain(g0, 0)
                fire_scatter(0, s0)
                drain(g1, 1)
                fire_scatter(1, s1)
                drain(s0, 0)
                load_idx(m + 2, 0)
                fire_gather(0, g0)
                return carry

            lax.fori_loop(0, NOP // 2 - 1, blk, 0)
            drain(s1, 1)
            load_idx(NOP - 1, 1)
            fire_gather(1, g1)
            drain(g0, 0)
            fire_scatter(0, s0)
            drain(g1, 1)
            fire_scatter(1, s1)
            drain(s0, 0)
            drain(s1, 1)
            plsc.subcore_barrier()
            for z in range(NFB):
                r0 = arow0 + z * FB
                pltpu.sync_copy(accum.at[pl.ds(r0, FB)], rows.at[pl.ds(0, FB)])
                pltpu.sync_copy(rows.at[pl.ds(0, FB)],
                                out.at[et, ck, pl.ds(r0, FB)])


@functools.partial(
    pl.kernel,
    out_type=jax.ShapeDtypeStruct((2, R, 16), jnp.float32),
    mesh=_mesh,
    scratch_types=[
        pltpu.VMEM((MACB * 128,), jnp.int32),
        pltpu.VMEM((MACB * 128, 16), jnp.float32),
        pltpu.VMEM((FB, 16), jnp.float32),
        pltpu.VMEM_SHARED((R, 16), jnp.float32),
    ],
    compiler_params=pltpu.CompilerParams(use_tc_tiling_on_sc=False),
)
def _count(dst_all, ones_hbm, zeros_hbm, out, didx, ones_v, zbuf, accum):
    ci = lax.axis_index("c")
    si = lax.axis_index("s")
    base_edge = si * SROWS * 128
    arow0 = si * TPR
    pltpu.sync_copy(ones_hbm, ones_v)
    pltpu.sync_copy(zeros_hbm, zbuf)
    for z in range(NFB):
        pltpu.sync_copy(zbuf.at[pl.ds(0, FB)],
                        accum.at[pl.ds(arow0 + z * FB, FB)])
    plsc.subcore_barrier()

    def blk(b, carry):
        e0 = base_edge + b * (MACB * 128)
        pltpu.sync_copy(dst_all.at[ci, pl.ds(e0, MACB * 128)], didx)
        pltpu.sync_copy(ones_v, accum.at[didx], add=True)
        return carry

    lax.fori_loop(0, NMAC, blk, 0)
    plsc.subcore_barrier()
    for z in range(NFB):
        r0 = arow0 + z * FB
        pltpu.sync_copy(accum.at[pl.ds(r0, FB)], zbuf)
        pltpu.sync_copy(zbuf, out.at[ci, pl.ds(r0, FB)])


def _tc_body(agg_ref, cnt_ref, x_ref, wl_ref, b_ref, wr_ref, g_ref, be_ref,
             o_ref):
    cnt = jnp.maximum(cnt_ref[0, :, 0:1], 1.0)
    acc = jnp.zeros((BN, C), jnp.float32)
    for ck in range(NCK):
        acc = acc + lax.dot_general(
            agg_ref[0, ck], wl_ref[0, :, ck * CH:(ck + 1) * CH],
            dimension_numbers=(((1,), (1,)), ((), ())),
            preferred_element_type=jnp.float32)
    y = (acc / cnt + b_ref[0]
         + lax.dot_general(x_ref[0], wr_ref[0],
                           dimension_numbers=(((1,), (1,)), ((), ())),
                           preferred_element_type=jnp.float32))
    mu = jnp.mean(y, axis=-1, keepdims=True)
    yc = y - mu
    var = jnp.mean(yc * yc, axis=-1, keepdims=True)
    z = yc * lax.rsqrt(var + 1e-5) * g_ref[0] + be_ref[0]
    o_ref[0] = jnp.maximum(z, 0.0)


def _tc_layer(agg, cnt, x, wl, bb, wr, g, be):
    return pl.pallas_call(
        _tc_body,
        grid=(2, N // BN),
        in_specs=[
            pl.BlockSpec((1, NCK, BN, CH), lambda t, i: (t, 0, i, 0)),
            pl.BlockSpec((1, BN, 16), lambda t, i: (t, i, 0)),
            pl.BlockSpec((1, BN, C), lambda t, i: (t, i, 0)),
            pl.BlockSpec((1, C, C), lambda t, i: (t, 0, 0)),
            pl.BlockSpec((1, 1, C), lambda t, i: (t, 0, 0)),
            pl.BlockSpec((1, C, C), lambda t, i: (t, 0, 0)),
            pl.BlockSpec((1, 1, C), lambda t, i: (t, 0, 0)),
            pl.BlockSpec((1, 1, C), lambda t, i: (t, 0, 0)),
        ],
        out_specs=pl.BlockSpec((1, BN, C), lambda t, i: (t, i, 0)),
        out_shape=jax.ShapeDtypeStruct((2, N, C), jnp.float32),
    )(agg, cnt, x, wl, bb, wr, g, be)


def _prep_edges(ei_u2i, ei_i2u):
    def one(ei, stack_off):
        srcp = jnp.concatenate(
            [ei[0], jnp.zeros((EPAD - E,), jnp.int32)])
        dstp = jnp.concatenate(
            [ei[1], jnp.full((EPAD - E,), N, jnp.int32)])
        s4 = (srcp + stack_off) * NCK
        s4 = s4[None, :] + jnp.arange(NCK, dtype=jnp.int32)[:, None]
        return s4, dstp

    s0, d0 = one(ei_u2i, N)
    s1, d1 = one(ei_i2u, 0)
    return jnp.stack([s0, s1]), jnp.stack([d0, d1])


def kernel(x_user, x_item, edge_index_u2i, edge_index_i2u, params):
    src4_all, dst_all = _prep_edges(edge_index_u2i, edge_index_i2u)
    ones16 = jnp.ones((MACB * 128, 16), jnp.float32)
    zeros16 = jnp.zeros((FB, 16), jnp.float32)
    zeros32 = jnp.zeros((2 * HM, CH), jnp.float32)
    cnt = _count(dst_all, ones16, zeros16)
    x = jnp.stack([x_item, x_user])
    for lp in params['layers']:
        wl = jnp.stack([lp['u2i']['W_l'], lp['i2u']['W_l']])
        bb = jnp.stack([lp['u2i']['b'], lp['i2u']['b']])[:, None, :]
        wr = jnp.stack([lp['u2i']['W_r'], lp['i2u']['W_r']])
        g = jnp.stack([lp['ln_item']['g'], lp['ln_user']['g']])[:, None, :]
        be = jnp.stack([lp['ln_item']['b'], lp['ln_user']['b']])[:, None, :]
        agg = _spmm(x.reshape(2 * N * NCK, CH), src4_all, dst_all, zeros32)
        x = _tc_layer(agg, cnt, x, wl, bb, wr, g, be)
    return x[1], x[0]
